# baseline (speedup 1.0000x reference)
import jax
import jax.numpy as jnp
from jax import lax
from jax.experimental import pallas as pl
from jax.experimental.pallas import tpu as pltpu
from jax.experimental.pallas import tpu_sc as plsc

S, E = 4096, 64
CAP = 80
NOISE_HI = 0x3F800000

_info = plsc.get_sparse_core_info()
NC, NS = _info.num_cores, _info.num_subcores
NW = NC * NS
TPW = S // NW
RB = 8
F = E * CAP
NCHUNK = TPW // RB


def _col_cumsum_excl(x):
    y = x
    sh = 1
    while sh < S:
        y = y + jnp.pad(y, ((sh, 0), (0, 0)))[:S, :]
        sh *= 2
    return y - x


def _route_kernel(x_ref, n_ref, wd_ref, te_ref, tc_ref, cnt_ref):
    x = x_ref[...]
    noise = n_ref[...]
    col = lax.broadcasted_iota(jnp.int32, (S, E), 1)

    row_max = jnp.max(x, axis=1, keepdims=True)
    is_max = x >= row_max
    e_idx = jnp.min(jnp.where(is_max, col, E), axis=1, keepdims=True)
    emask = col == e_idx
    z = jnp.sum(jnp.exp(x - row_max), axis=1, keepdims=True)
    prob = 1.0 / z

    cnt_ref[...] = jnp.sum(emask.astype(jnp.int32), axis=0, keepdims=True)

    keys = lax.bitcast_convert_type(jnp.where(emask, noise, 0.0), jnp.int32)

    def body(_, carry):
        lo, hi = carry
        mid = lo + (hi - lo + 1) // 2
        cge = jnp.sum((keys >= mid).astype(jnp.int32), axis=0, keepdims=True)
        ge = cge >= CAP
        return jnp.where(ge, mid, lo), jnp.where(ge, hi, mid - 1)

    lo0 = jnp.zeros((1, E), jnp.int32)
    hi0 = jnp.full((1, E), NOISE_HI, jnp.int32)
    vcap, _ = lax.fori_loop(0, 31, body, (lo0, hi0))

    n_gt = jnp.sum((keys > vcap).astype(jnp.int32), axis=0, keepdims=True)
    ties_needed = CAP - n_gt
    is_tie = keys == vcap
    tie_rank = _col_cumsum_excl(is_tie.astype(jnp.int32))
    sel = (keys > vcap) | (is_tie & (tie_rank < ties_needed))
    disp = emask & sel

    loc_x = _col_cumsum_excl(disp.astype(jnp.int32))
    loc = jnp.sum(jnp.where(disp, loc_x, 0), axis=1, keepdims=True)
    disp_t = jnp.any(disp, axis=1, keepdims=True)

    wd_ref[...] = jnp.where(disp_t, prob, 0.0)
    te_ref[...] = jnp.where(disp_t, e_idx, -1)
    tc_ref[...] = loc


def _sc_fill(wd_hbm, te_hbm, tc_hbm, cwf_hbm, mw_hbm, wdv, tev, tcv, zf, zm):
    wid = lax.axis_index("s") * NC + lax.axis_index("c")
    base = wid * TPW
    pltpu.sync_copy(wd_hbm.at[pl.ds(base, TPW)], wdv)
    pltpu.sync_copy(te_hbm.at[pl.ds(base, TPW)], tev)
    pltpu.sync_copy(tc_hbm.at[pl.ds(base, TPW)], tcv)

    lane = lax.broadcasted_iota(jnp.int32, (16,), 0)
    z16 = jnp.zeros((16,), jnp.float32)
    zi16 = jnp.zeros((16,), jnp.int32)

    def zero_f(i, _):
        zf[i // (F // 16), pl.ds((i % (F // 16)) * 16, 16)] = z16
        return 0

    def zero_m(i, _):
        zm[i // (F // 64), pl.ds((i % (F // 64)) * 16, 16)] = zi16
        return 0

    lax.fori_loop(0, RB * (F // 16), zero_f, 0)
    lax.fori_loop(0, RB * (F // 64), zero_m, 0)

    for ci in range(NCHUNK):
        g, h = divmod(ci, 2)
        te16 = tev[pl.ds(16 * g, 16)]
        tc16 = tcv[pl.ds(16 * g, 16)]
        wd16 = wdv[pl.ds(16 * g, 16)]
        msk = ((lane >= 8 * h) & (lane < 8 * h + 8)) & (te16 >= 0)
        r16 = jnp.clip(lane - 8 * h, 0, RB - 1)
        j16 = jnp.clip(te16 * CAP + tc16, 0, F - 1)
        plsc.store_scatter(zf, [r16, j16], wd16, mask=msk)
        widx = j16 // 4
        wval = jnp.left_shift(jnp.ones((16,), jnp.int32), 8 * (j16 % 4))
        plsc.store_scatter(zm, [r16, widx], wval, mask=msk)
        pltpu.sync_copy(zf, cwf_hbm.at[pl.ds(base + ci * RB, RB)])
        pltpu.sync_copy(zm, mw_hbm.at[pl.ds(base + ci * RB, RB)])
        plsc.store_scatter(zf, [r16, j16], z16, mask=msk)
        plsc.store_scatter(zm, [r16, widx], zi16, mask=msk)


def kernel(inputs, rand_noise):
    wd, te, tc, cnt = pl.pallas_call(
        _route_kernel,
        out_shape=[
            jax.ShapeDtypeStruct((S, 1), jnp.float32),
            jax.ShapeDtypeStruct((S, 1), jnp.int32),
            jax.ShapeDtypeStruct((S, 1), jnp.int32),
            jax.ShapeDtypeStruct((1, E), jnp.int32),
        ],
    )(inputs, rand_noise)

    mesh = plsc.VectorSubcoreMesh(core_axis_name="c", subcore_axis_name="s")
    cwf, mw = pl.kernel(
        _sc_fill,
        mesh=mesh,
        out_type=[
            jax.ShapeDtypeStruct((S, F), jnp.float32),
            jax.ShapeDtypeStruct((S, F // 4), jnp.int32),
        ],
        scratch_types=[
            pltpu.VMEM((TPW,), jnp.float32),
            pltpu.VMEM((TPW,), jnp.int32),
            pltpu.VMEM((TPW,), jnp.int32),
            pltpu.VMEM((RB, F), jnp.float32),
            pltpu.VMEM((RB, F // 4), jnp.int32),
        ],
        compiler_params=pltpu.CompilerParams(needs_layout_passes=False),
    )(wd.reshape(S), te.reshape(S), tc.reshape(S))

    cw = cwf.reshape(S, E, CAP)
    m = lax.bitcast_convert_type(mw, jnp.int8).reshape(S, E, CAP)
    return cw, m.astype(jnp.bool_), cnt.reshape(E)


# speedup vs baseline: 1.7045x; 1.7045x over previous
import jax
import jax.numpy as jnp
from jax import lax
from jax.experimental import pallas as pl

S, E = 4096, 64
CAP = 80
NOISE_HI = 0x3F800000
TS = 256


def _col_cumsum_excl(x):
    y = x
    sh = 1
    while sh < S:
        y = y + jnp.pad(y, ((sh, 0), (0, 0)))[:S, :]
        sh *= 2
    return y - x


def _route_kernel(x_ref, n_ref, wd_ref, tgt_ref, cnt_ref):
    x = x_ref[...]
    noise = n_ref[...]
    col = lax.broadcasted_iota(jnp.int32, (S, E), 1)

    row_max = jnp.max(x, axis=1, keepdims=True)
    is_max = x >= row_max
    e_idx = jnp.min(jnp.where(is_max, col, E), axis=1, keepdims=True)
    emask = col == e_idx
    z = jnp.sum(jnp.exp(x - row_max), axis=1, keepdims=True)
    prob = 1.0 / z

    cnt_ref[...] = jnp.sum(emask.astype(jnp.int32), axis=0, keepdims=True)

    keys = lax.bitcast_convert_type(jnp.where(emask, noise, 0.0), jnp.int32)

    def body(_, carry):
        lo, hi = carry
        mid = lo + (hi - lo + 1) // 2
        cge = jnp.sum((keys >= mid).astype(jnp.int32), axis=0, keepdims=True)
        ge = cge >= CAP
        return jnp.where(ge, mid, lo), jnp.where(ge, hi, mid - 1)

    lo0 = jnp.zeros((1, E), jnp.int32)
    hi0 = jnp.full((1, E), NOISE_HI, jnp.int32)
    vcap, _ = lax.fori_loop(0, 31, body, (lo0, hi0))

    n_gt = jnp.sum((keys > vcap).astype(jnp.int32), axis=0, keepdims=True)
    ties_needed = CAP - n_gt
    is_tie = keys == vcap
    tie_rank = _col_cumsum_excl(is_tie.astype(jnp.int32))
    sel = (keys > vcap) | (is_tie & (tie_rank < ties_needed))
    disp = emask & sel

    loc_x = _col_cumsum_excl(disp.astype(jnp.int32))
    loc = jnp.sum(jnp.where(disp, loc_x, 0), axis=1, keepdims=True)
    disp_t = jnp.any(disp, axis=1, keepdims=True)

    wd_ref[...] = jnp.where(disp_t, prob, 0.0)
    tgt_ref[...] = jnp.where(disp_t, e_idx * CAP + loc, -1)


def _fill_kernel(wd_ref, tgt_ref, cw_ref, m_ref):
    j = lax.broadcasted_iota(jnp.int32, (TS, E * CAP), 1)
    hit = j == tgt_ref[...]
    cw_ref[...] = jnp.where(hit, wd_ref[...], 0.0)
    m_ref[...] = hit


def kernel(inputs, rand_noise):
    wd, tgt, cnt = pl.pallas_call(
        _route_kernel,
        out_shape=[
            jax.ShapeDtypeStruct((S, 1), jnp.float32),
            jax.ShapeDtypeStruct((S, 1), jnp.int32),
            jax.ShapeDtypeStruct((1, E), jnp.int32),
        ],
    )(inputs, rand_noise)
    cw, m = pl.pallas_call(
        _fill_kernel,
        grid=(S // TS,),
        in_specs=[
            pl.BlockSpec((TS, 1), lambda i: (i, 0)),
            pl.BlockSpec((TS, 1), lambda i: (i, 0)),
        ],
        out_specs=[
            pl.BlockSpec((TS, E * CAP), lambda i: (i, 0)),
            pl.BlockSpec((TS, E * CAP), lambda i: (i, 0)),
        ],
        out_shape=[
            jax.ShapeDtypeStruct((S, E * CAP), jnp.float32),
            jax.ShapeDtypeStruct((S, E * CAP), jnp.bool_),
        ],
    )(wd, tgt)
    return cw.reshape(S, E, CAP), m.reshape(S, E, CAP), cnt.reshape(E)
